# baseline (speedup 1.0000x reference)
import numpy as np
import jax
import jax.numpy as jnp
from jax.experimental import pallas as pl
from jax.experimental.pallas import tpu as pltpu

B = 4096
IN = 256
H = 768
D = 32
NC = 8
CPC = 128
SH = D // 2
ND = NC * D
NSH = NC * SH
BB = 1024
NBLK = B // BB

_VQ_SCALE = np.float32(1.25 / (B * D))


def _dot(a, b):
    return jax.lax.dot_general(a.astype(jnp.bfloat16), b.astype(jnp.bfloat16),
                               (((1,), (0,)), ((), ())),
                               preferred_element_type=jnp.float32)


def _dotx(a, b):
    return jax.lax.dot_general(a, b, (((1,), (0,)), ((), ())),
                               precision=jax.lax.Precision.HIGHEST,
                               preferred_element_type=jnp.float32)


_ERF_T = [np.float32(c) for c in
          ("7.85386146e-05", "-0.000801019371", "0.00518832775", "-0.0268538129",
           "0.112835854", "-0.37612626", "1.12837911")]
_ERFC_P = [np.float32(c) for c in
           ("0.0232682", "-0.138703942", "0.368742466", "-0.582473278",
            "0.621000469", "-0.494451523", "0.340488", "-0.274112701",
            "0.563825965")]
_ERFC_R = [np.float32(c) for c in
           ("-10.477664", "12.9772", "-7.49551868", "2.92101908", "-1.01526523",
            "0.42184633", "-0.282076746", "0.564189494")]


def _erfc(x):
    one = np.float32(1.0)
    x2 = x * x
    absx = jnp.abs(x)
    pt = x2 * _ERF_T[0] + _ERF_T[1]
    for c in _ERF_T[2:]:
        pt = pt * x2 + c
    res_lt1 = one - x * pt
    y = one / x2
    pp = y * _ERFC_P[0] + _ERFC_P[1]
    for c in _ERFC_P[2:]:
        pp = pp * y + c
    pr = y * _ERFC_R[0] + _ERFC_R[1]
    for c in _ERFC_R[2:]:
        pr = pr * y + c
    z = jnp.exp(-x2)
    base = z * (one / absx)
    val = base * jnp.where(absx < np.float32(2.0), pp, pr)
    val = jnp.where(-x2 < np.float32(-88.7228394), np.float32(0.0), val)
    val = jnp.where(x < np.float32(0.0), np.float32(2.0) - val, val)
    return jnp.where(absx < one, res_lt1, val)


def _gelu(t):
    return 0.5 * t * _erfc(-t * np.sqrt(0.5).astype(np.float32))


def _fused_kernel(x_ref, w1_ref, b1_ref, w2_ref, b2_ref, wk_ref, bk_ref,
                  cq_ref, wv_ref, bv_ref, cb_ref, cbt_ref,
                  ws1bd_ref, bs1t_ref, ws2bd_ref, bs2t_ref,
                  kchart_ref, kcode_ref, zn_ref, ztex_ref, rw_ref, zgeo_ref,
                  vq_ref, idx_ref, znall_ref):
    x = x_ref[...]
    h1 = _gelu(_dot(x, w1_ref[...]) + b1_ref[...])
    feats = _gelu(_dot(h1, w2_ref[...]) + b2_ref[...])
    k = _dot(feats, wk_ref[...]) + bk_ref[...]
    scores = _dot(k, cq_ref[...]) / np.sqrt(float(H)).astype(np.float32)

    m = jnp.max(scores, axis=-1, keepdims=True)
    e = jnp.exp(scores - m)
    w = e / jnp.sum(e, axis=-1, keepdims=True)
    rw_ref[...] = w

    iota8 = jax.lax.broadcasted_iota(jnp.int32, (BB, NC), 1)
    wmax = jnp.max(w, axis=-1, keepdims=True)
    kchart = jnp.min(jnp.where(w == wmax, iota8, NC), axis=-1, keepdims=True)
    kchart_ref[...] = kchart

    v = _dot(feats, wv_ref[...]) + bv_ref[...]

    g = _dotx(v, cbt_ref[...])
    cbt = cbt_ref[...]
    cn = _dotx(jnp.ones((1, D), jnp.float32), cbt * cbt)
    t = cn - 2.0 * g
    iota128 = jax.lax.broadcasted_iota(jnp.int32, (BB, CPC), 1)

    kcode = jnp.zeros((BB, 1), dtype=jnp.int32)
    zq_parts = []
    for c in range(NC):
        tc = t[:, c * CPC:(c + 1) * CPC]
        tmin = jnp.min(tc, axis=-1, keepdims=True)
        idx_c = jnp.min(jnp.where(tc == tmin, iota128, CPC), axis=-1, keepdims=True)
        idx_ref[:, c:c + 1] = idx_c
        kcode = kcode + jnp.where(kchart == c, idx_c, 0)
        onehot = (iota128 == idx_c).astype(jnp.float32)
        zq_parts.append(_dotx(onehot, cb_ref[c]))
    kcode_ref[...] = kcode

    zq_all = jnp.concatenate(zq_parts, axis=1)

    rep_v = (jax.lax.broadcasted_iota(jnp.int32, (D, ND), 0)
             == jax.lax.broadcasted_iota(jnp.int32, (D, ND), 1) % D
             ).astype(jnp.float32)
    rep_w = (jax.lax.broadcasted_iota(jnp.int32, (NC, ND), 0)
             == jax.lax.broadcasted_iota(jnp.int32, (NC, ND), 1) // D
             ).astype(jnp.float32)
    tile_eye = (jax.lax.broadcasted_iota(jnp.int32, (ND, D), 0) % D
                == jax.lax.broadcasted_iota(jnp.int32, (ND, D), 1)
                ).astype(jnp.float32)
    v_tiled = _dotx(v, rep_v)
    w_rep = _dotx(w, rep_w)

    delta_all = v_tiled - zq_all
    loss = jnp.sum(delta_all * delta_all * w_rep, keepdims=True) * _VQ_SCALE

    hidden = _gelu(_dot(delta_all, ws1bd_ref[...]) + bs1t_ref[...])
    zn_all = _dot(hidden, ws2bd_ref[...]) + bs2t_ref[...]
    znall_ref[...] = zn_all

    zq_b = _dotx(zq_all * w_rep, tile_eye)
    zn_b = _dotx(zn_all * w_rep, tile_eye)

    zn_ref[...] = zn_b
    ztex_ref[...] = (v - zq_b) - zn_b
    zgeo_ref[...] = (v + (zq_b - v)) + zn_b

    @pl.when(pl.program_id(0) == 0)
    def _init():
        vq_ref[...] = jnp.zeros((1, 1), dtype=jnp.float32)
    vq_ref[...] += loss


def kernel(x, W1, b1, W2, b2, Wk, bk, chart_queries, Wv, bv, codebook,
           Ws1, bs1, Ws2, bs2):
    cbt = codebook.reshape(NC * CPC, D).T
    cnd = jnp.arange(ND)[None, :]
    rnd = jnp.arange(ND)[:, None]
    cnsh = jnp.arange(NSH)[None, :]
    ws1_bd = jnp.where(rnd // D == cnsh // SH, jnp.tile(Ws1, (NC, NC)), 0.0)
    rnsh = jnp.arange(NSH)[:, None]
    ws2_bd = jnp.where(rnsh // SH == cnd // D, jnp.tile(Ws2, (NC, NC)), 0.0)
    bs1_t = jnp.tile(bs1, NC)[None, :]
    bs2_t = jnp.tile(bs2, NC)[None, :]

    bf = jnp.bfloat16
    x_b = x.astype(bf)
    W1_b, W2_b, Wk_b, Wv_b = (W1.astype(bf), W2.astype(bf), Wk.astype(bf),
                              Wv.astype(bf))
    cq_b = chart_queries.T.astype(bf)
    ws1_b, ws2_b = ws1_bd.astype(bf), ws2_bd.astype(bf)

    full = lambda *shape: pl.BlockSpec(shape, lambda i: (0,) * len(shape))
    row = lambda *shape: pl.BlockSpec(shape, lambda i: (i,) + (0,) * (len(shape) - 1))

    out_shapes = (
        jax.ShapeDtypeStruct((B, 1), jnp.int32),
        jax.ShapeDtypeStruct((B, 1), jnp.int32),
        jax.ShapeDtypeStruct((B, D), jnp.float32),
        jax.ShapeDtypeStruct((B, D), jnp.float32),
        jax.ShapeDtypeStruct((B, NC), jnp.float32),
        jax.ShapeDtypeStruct((B, D), jnp.float32),
        jax.ShapeDtypeStruct((1, 1), jnp.float32),
        jax.ShapeDtypeStruct((B, NC), jnp.int32),
        jax.ShapeDtypeStruct((B, ND), jnp.float32),
    )
    in_specs = [
        row(BB, IN),
        full(IN, H), full(1, H), full(H, H), full(1, H), full(H, H), full(1, H),
        full(H, NC), full(H, D), full(1, D), full(NC, CPC, D), full(D, NC * CPC),
        full(ND, NSH), full(1, NSH), full(NSH, ND), full(1, ND),
    ]
    out_specs = (
        row(BB, 1), row(BB, 1), row(BB, D), row(BB, D), row(BB, NC), row(BB, D),
        full(1, 1), row(BB, NC), row(BB, ND),
    )
    outs = pl.pallas_call(
        _fused_kernel,
        grid=(NBLK,),
        in_specs=in_specs,
        out_specs=out_specs,
        out_shape=out_shapes,
    )(x_b, W1_b, b1[None, :], W2_b, b2[None, :], Wk_b, bk[None, :],
      cq_b, Wv_b, bv[None, :], codebook, cbt,
      ws1_b, bs1_t, ws2_b, bs2_t)

    kchart, kcode, z_n, z_tex, rw, z_geo, vq, idx, znall = outs
    return (kchart[:, 0], kcode[:, 0], z_n, z_tex, rw, z_geo, vq[0, 0], idx,
            znall.reshape(B, NC, D))


# speedup vs baseline: 1.2095x; 1.2095x over previous
import numpy as np
import jax
import jax.numpy as jnp
from jax.experimental import pallas as pl
from jax.experimental.pallas import tpu as pltpu

B = 4096
IN = 256
H = 768
D = 32
NC = 8
CPC = 128
SH = D // 2
ND = NC * D
NSH = NC * SH
BB = 1024
NBLK = B // BB

_VQ_SCALE = np.float32(1.25 / (B * D))


def _dot(a, b):
    return jax.lax.dot_general(a.astype(jnp.bfloat16), b.astype(jnp.bfloat16),
                               (((1,), (0,)), ((), ())),
                               preferred_element_type=jnp.float32)


def _dotx(a, b):
    return jax.lax.dot_general(a, b, (((1,), (0,)), ((), ())),
                               preferred_element_type=jnp.float32)


_ERF_T = [np.float32(c) for c in
          ("7.85386146e-05", "-0.000801019371", "0.00518832775", "-0.0268538129",
           "0.112835854", "-0.37612626", "1.12837911")]
_ERFC_P = [np.float32(c) for c in
           ("0.0232682", "-0.138703942", "0.368742466", "-0.582473278",
            "0.621000469", "-0.494451523", "0.340488", "-0.274112701",
            "0.563825965")]
_ERFC_R = [np.float32(c) for c in
           ("-10.477664", "12.9772", "-7.49551868", "2.92101908", "-1.01526523",
            "0.42184633", "-0.282076746", "0.564189494")]


def _erfc(x):
    one = np.float32(1.0)
    x2 = x * x
    absx = jnp.abs(x)
    pt = x2 * _ERF_T[0] + _ERF_T[1]
    for c in _ERF_T[2:]:
        pt = pt * x2 + c
    res_lt1 = one - x * pt
    y = one / x2
    pp = y * _ERFC_P[0] + _ERFC_P[1]
    for c in _ERFC_P[2:]:
        pp = pp * y + c
    pr = y * _ERFC_R[0] + _ERFC_R[1]
    for c in _ERFC_R[2:]:
        pr = pr * y + c
    z = jnp.exp(-x2)
    base = z * (one / absx)
    val = base * jnp.where(absx < np.float32(2.0), pp, pr)
    val = jnp.where(-x2 < np.float32(-88.7228394), np.float32(0.0), val)
    val = jnp.where(x < np.float32(0.0), np.float32(2.0) - val, val)
    return jnp.where(absx < one, res_lt1, val)


def _gelu(t):
    return 0.5 * t * _erfc(-t * np.sqrt(0.5).astype(np.float32))


def _fused_kernel(x_ref, w1_ref, b1_ref, w2_ref, b2_ref, wk_ref, bk_ref,
                  cq_ref, wv_ref, bv_ref, cb_ref, cbt_ref,
                  ws1bd_ref, bs1t_ref, ws2bd_ref, bs2t_ref,
                  kchart_ref, kcode_ref, zn_ref, ztex_ref, rw_ref, zgeo_ref,
                  vq_ref, idx_ref, znall_ref):
    x = x_ref[...]
    h1 = _gelu(_dot(x, w1_ref[...]) + b1_ref[...])
    feats = _gelu(_dot(h1, w2_ref[...]) + b2_ref[...])
    k = _dot(feats, wk_ref[...]) + bk_ref[...]
    scores = _dot(k, cq_ref[...]) / np.sqrt(float(H)).astype(np.float32)

    m = jnp.max(scores, axis=-1, keepdims=True)
    e = jnp.exp(scores - m)
    w = e / jnp.sum(e, axis=-1, keepdims=True)
    rw_ref[...] = w

    iota8 = jax.lax.broadcasted_iota(jnp.int32, (BB, NC), 1)
    wmax = jnp.max(w, axis=-1, keepdims=True)
    kchart = jnp.min(jnp.where(w == wmax, iota8, NC), axis=-1, keepdims=True)
    kchart_ref[...] = kchart

    v = _dot(feats, wv_ref[...]) + bv_ref[...]

    g = _dotx(v, cbt_ref[...])
    cbt = cbt_ref[...]
    cn = _dotx(jnp.ones((1, D), jnp.float32), cbt * cbt)
    t = cn - 2.0 * g
    iota128 = jax.lax.broadcasted_iota(jnp.int32, (BB, CPC), 1)

    kcode = jnp.zeros((BB, 1), dtype=jnp.int32)
    zq_parts = []
    for c in range(NC):
        tc = t[:, c * CPC:(c + 1) * CPC]
        tmin = jnp.min(tc, axis=-1, keepdims=True)
        idx_c = jnp.min(jnp.where(tc == tmin, iota128, CPC), axis=-1, keepdims=True)
        idx_ref[:, c:c + 1] = idx_c
        kcode = kcode + jnp.where(kchart == c, idx_c, 0)
        onehot = (iota128 == idx_c).astype(jnp.float32)
        zq_parts.append(_dotx(onehot, cb_ref[c]))
    kcode_ref[...] = kcode

    zq_all = jnp.concatenate(zq_parts, axis=1)

    rep_v = (jax.lax.broadcasted_iota(jnp.int32, (D, ND), 0)
             == jax.lax.broadcasted_iota(jnp.int32, (D, ND), 1) % D
             ).astype(jnp.float32)
    rep_w = (jax.lax.broadcasted_iota(jnp.int32, (NC, ND), 0)
             == jax.lax.broadcasted_iota(jnp.int32, (NC, ND), 1) // D
             ).astype(jnp.float32)
    tile_eye = (jax.lax.broadcasted_iota(jnp.int32, (ND, D), 0) % D
                == jax.lax.broadcasted_iota(jnp.int32, (ND, D), 1)
                ).astype(jnp.float32)
    v_tiled = _dotx(v, rep_v)
    w_rep = _dotx(w, rep_w)

    delta_all = v_tiled - zq_all
    loss = jnp.sum(delta_all * delta_all * w_rep, keepdims=True) * _VQ_SCALE

    hidden = _gelu(_dot(delta_all, ws1bd_ref[...]) + bs1t_ref[...])
    zn_all = _dot(hidden, ws2bd_ref[...]) + bs2t_ref[...]
    znall_ref[...] = zn_all

    zq_b = _dotx(zq_all * w_rep, tile_eye)
    zn_b = _dotx(zn_all * w_rep, tile_eye)

    zn_ref[...] = zn_b
    ztex_ref[...] = (v - zq_b) - zn_b
    zgeo_ref[...] = (v + (zq_b - v)) + zn_b

    @pl.when(pl.program_id(0) == 0)
    def _init():
        vq_ref[...] = jnp.zeros((1, 1), dtype=jnp.float32)
    vq_ref[...] += loss


def kernel(x, W1, b1, W2, b2, Wk, bk, chart_queries, Wv, bv, codebook,
           Ws1, bs1, Ws2, bs2):
    cbt = codebook.reshape(NC * CPC, D).T
    cnd = jnp.arange(ND)[None, :]
    rnd = jnp.arange(ND)[:, None]
    cnsh = jnp.arange(NSH)[None, :]
    ws1_bd = jnp.where(rnd // D == cnsh // SH, jnp.tile(Ws1, (NC, NC)), 0.0)
    rnsh = jnp.arange(NSH)[:, None]
    ws2_bd = jnp.where(rnsh // SH == cnd // D, jnp.tile(Ws2, (NC, NC)), 0.0)
    bs1_t = jnp.tile(bs1, NC)[None, :]
    bs2_t = jnp.tile(bs2, NC)[None, :]

    bf = jnp.bfloat16
    x_b = x.astype(bf)
    W1_b, W2_b, Wk_b, Wv_b = (W1.astype(bf), W2.astype(bf), Wk.astype(bf),
                              Wv.astype(bf))
    cq_b = chart_queries.T.astype(bf)
    ws1_b, ws2_b = ws1_bd.astype(bf), ws2_bd.astype(bf)

    full = lambda *shape: pl.BlockSpec(shape, lambda i: (0,) * len(shape))
    row = lambda *shape: pl.BlockSpec(shape, lambda i: (i,) + (0,) * (len(shape) - 1))

    out_shapes = (
        jax.ShapeDtypeStruct((B, 1), jnp.int32),
        jax.ShapeDtypeStruct((B, 1), jnp.int32),
        jax.ShapeDtypeStruct((B, D), jnp.float32),
        jax.ShapeDtypeStruct((B, D), jnp.float32),
        jax.ShapeDtypeStruct((B, NC), jnp.float32),
        jax.ShapeDtypeStruct((B, D), jnp.float32),
        jax.ShapeDtypeStruct((1, 1), jnp.float32),
        jax.ShapeDtypeStruct((B, NC), jnp.int32),
        jax.ShapeDtypeStruct((B, ND), jnp.float32),
    )
    in_specs = [
        row(BB, IN),
        full(IN, H), full(1, H), full(H, H), full(1, H), full(H, H), full(1, H),
        full(H, NC), full(H, D), full(1, D), full(NC, CPC, D), full(D, NC * CPC),
        full(ND, NSH), full(1, NSH), full(NSH, ND), full(1, ND),
    ]
    out_specs = (
        row(BB, 1), row(BB, 1), row(BB, D), row(BB, D), row(BB, NC), row(BB, D),
        full(1, 1), row(BB, NC), row(BB, ND),
    )
    outs = pl.pallas_call(
        _fused_kernel,
        grid=(NBLK,),
        in_specs=in_specs,
        out_specs=out_specs,
        out_shape=out_shapes,
    )(x_b, W1_b, b1[None, :], W2_b, b2[None, :], Wk_b, bk[None, :],
      cq_b, Wv_b, bv[None, :], codebook, cbt,
      ws1_b, bs1_t, ws2_b, bs2_t)

    kchart, kcode, z_n, z_tex, rw, z_geo, vq, idx, znall = outs
    return (kchart[:, 0], kcode[:, 0], z_n, z_tex, rw, z_geo, vq[0, 0], idx,
            znall.reshape(B, NC, D))
